# baseline (speedup 1.0000x reference)
import jax
import jax.numpy as jnp
from jax.experimental import pallas as pl
from jax.experimental.pallas import tpu as pltpu
from jax.experimental.pallas import tpu_sc as plsc

BATCH = 16384
DIM = 64
WINDOW = 128
LANES = 16


def kernel(user, item, user_table, item_table):
    user2 = user.reshape(1, BATCH)
    item2 = item.reshape(1, BATCH)

    mesh = plsc.VectorSubcoreMesh(core_axis_name="core",
                                  subcore_axis_name="subcore")

    @pl.kernel(
        out_type=jax.ShapeDtypeStruct((BATCH, DIM), jnp.float32),
        mesh=mesh,
        compiler_params=pltpu.CompilerParams(use_tc_tiling_on_sc=False),
        scratch_types=[
            pltpu.VMEM((WINDOW, DIM), jnp.float32),
            pltpu.VMEM((WINDOW, DIM), jnp.float32),
            pltpu.SemaphoreType.DMA,
            pltpu.SemaphoreType.DMA,
        ],
    )
    def sc_kernel(u_hbm, i_hbm, ut_hbm, it_hbm, o_hbm, ubuf, ibuf, sem_u, sem_i):
        def body(u_idx, i_idx, o_vmem):
            cp_u = pltpu.async_copy(ut_hbm.at[u_idx.at[0]], ubuf, sem_u)
            cp_i = pltpu.async_copy(it_hbm.at[i_idx.at[0]], ibuf, sem_i)
            cp_u.wait()
            cp_i.wait()

            @pl.loop(0, WINDOW)
            def _(r):
                @pl.loop(0, DIM, step=LANES)
                def _(c):
                    slc = (pl.ds(r, 1), pl.ds(c, LANES))
                    o_vmem.at[*slc][...] = ubuf.at[*slc][...] * ibuf.at[*slc][...]

        pltpu.emit_pipeline(
            body,
            grid=(BATCH // WINDOW,),
            in_specs=[
                pl.BlockSpec((1, WINDOW), lambda i: (0, i)),
                pl.BlockSpec((1, WINDOW), lambda i: (0, i)),
            ],
            out_specs=[pl.BlockSpec((WINDOW, DIM), lambda i: (i, 0))],
            core_axis_name=("core", "subcore"),
            dimension_semantics=(pltpu.PARALLEL,),
        )(u_hbm, i_hbm, o_hbm)

    return sc_kernel(user2, item2, user_table, item_table)
